# baseline (speedup 1.0000x reference)
import functools

import jax
import jax.numpy as jnp
from jax import lax
from jax.experimental import pallas as pl
from jax.experimental.pallas import tpu as pltpu
from jax.experimental.pallas import tpu_sc as plsc

_VOCAB = 100000
_EMBED = 16
_CTX = 20
_BATCH = 1024

_NC = 2
_NS = 16
_NW = _NC * _NS
_ROWS_W = _BATCH // _NW
_IDX_W = _ROWS_W * _CTX
_CHUNK = 128
_NCHUNK = _IDX_W // _CHUNK

@functools.lru_cache(maxsize=1)
def _build_pool_sc():
    mesh = plsc.VectorSubcoreMesh(core_axis_name="c", subcore_axis_name="s")

    @functools.partial(
        pl.kernel,
        mesh=mesh,
        out_type=jax.ShapeDtypeStruct((_BATCH, _EMBED), jnp.float32),
        scratch_types=[
            pltpu.VMEM((_IDX_W,), jnp.int32),
            pltpu.VMEM((_IDX_W, _EMBED), jnp.float32),
            pltpu.VMEM((_ROWS_W, _EMBED), jnp.float32),
            pltpu.SemaphoreType.DMA,
        ],
        compiler_params=pltpu.CompilerParams(use_tc_tiling_on_sc=False),
    )
    def pool_sc(idx_hbm, table_hbm, out_hbm, idx_v, rows_v, out_v, sem):
        wid = lax.axis_index("s") * _NC + lax.axis_index("c")
        pltpu.sync_copy(idx_hbm.at[pl.ds(wid * _IDX_W, _IDX_W)], idx_v)
        copies = []
        for j in range(_NCHUNK):
            copies.append(
                pltpu.async_copy(
                    table_hbm.at[idx_v.at[pl.ds(j * _CHUNK, _CHUNK)]],
                    rows_v.at[pl.ds(j * _CHUNK, _CHUNK)],
                    sem,
                )
            )
        for c in copies:
            c.wait()
        inv = jnp.float32(1.0 / _CTX)
        for r in range(_ROWS_W):
            acc = rows_v[r * _CTX]
            for t in range(1, _CTX):
                acc = acc + rows_v[r * _CTX + t]
            out_v[r] = acc * inv
        pltpu.sync_copy(out_v, out_hbm.at[pl.ds(wid * _ROWS_W, _ROWS_W)])

    return pool_sc


_PCOL = 16384
_PGRID = (_VOCAB + _PCOL - 1) // _PCOL


def _pack_body(wt_ref, out_ref):
    y = wt_ref[...].T.reshape(_PCOL // 8, 8, _EMBED)
    for s in range(8):
        out_ref[:, s * _EMBED:(s + 1) * _EMBED] = y[:, s, :]


def _pack_table(Wt_view):
    return pl.pallas_call(
        _pack_body,
        grid=(_PGRID,),
        in_specs=[pl.BlockSpec((_EMBED, _PCOL), lambda i: (0, i))],
        out_specs=pl.BlockSpec((_PCOL * _EMBED // 128, 128), lambda i: (i, 0)),
        out_shape=jax.ShapeDtypeStruct((_VOCAB * _EMBED // 128, 128),
                                       jnp.float32),
    )(Wt_view)


_VB = 4096
_GRID = (_VOCAB + _VB - 1) // _VB


def _proj_body(wt_ref, emb_ref, b_ref, out_ref):
    out_ref[...] = (
        lax.dot_general(
            wt_ref[...],
            emb_ref[...],
            dimension_numbers=(((0,), (1,)), ((), ())),
            preferred_element_type=jnp.float32,
        )
        + b_ref[...].T
    )


def _project_t(pooled, Wt, b2):
    return pl.pallas_call(
        _proj_body,
        grid=(_GRID,),
        in_specs=[
            pl.BlockSpec((_EMBED, _VB), lambda i: (0, i)),
            pl.BlockSpec((_BATCH, _EMBED), lambda i: (0, 0)),
            pl.BlockSpec((1, _VB), lambda i: (0, i)),
        ],
        out_specs=pl.BlockSpec((_VB, _BATCH), lambda i: (i, 0)),
        out_shape=jax.ShapeDtypeStruct((_VOCAB, _BATCH), jnp.float32),
    )(Wt, pooled, b2)


def kernel(context_words, target_word, emb_table, W, b):
    del target_word
    idx = context_words.astype(jnp.int32).reshape(_BATCH * _CTX)
    packed = _pack_table(emb_table.T)
    pooled = _build_pool_sc()(idx, packed.reshape(_VOCAB, _EMBED))
    logits_t = _project_t(pooled, W.T, b.reshape(1, _VOCAB))
    return logits_t.T


# speedup vs baseline: 1.0102x; 1.0102x over previous
import functools

import jax
import jax.numpy as jnp
from jax import lax
from jax.experimental import pallas as pl
from jax.experimental.pallas import tpu as pltpu
from jax.experimental.pallas import tpu_sc as plsc

_VOCAB = 100000
_EMBED = 16
_CTX = 20
_BATCH = 1024

_NC = 2
_NS = 16
_NW = _NC * _NS
_ROWS_W = _BATCH // _NW
_IDX_W = _ROWS_W * _CTX
_CHUNK = 128
_NCHUNK = _IDX_W // _CHUNK

@functools.lru_cache(maxsize=1)
def _build_pool_sc():
    mesh = plsc.VectorSubcoreMesh(core_axis_name="c", subcore_axis_name="s")

    @functools.partial(
        pl.kernel,
        mesh=mesh,
        out_type=jax.ShapeDtypeStruct((_BATCH, _EMBED), jnp.float32),
        scratch_types=[
            pltpu.VMEM((_IDX_W,), jnp.int32),
            pltpu.VMEM((_IDX_W, _EMBED), jnp.float32),
            pltpu.VMEM((_ROWS_W, _EMBED), jnp.float32),
            pltpu.SemaphoreType.DMA,
        ],
        compiler_params=pltpu.CompilerParams(use_tc_tiling_on_sc=False),
    )
    def pool_sc(idx_hbm, table_hbm, out_hbm, idx_v, rows_v, out_v, sem):
        wid = lax.axis_index("s") * _NC + lax.axis_index("c")
        pltpu.sync_copy(idx_hbm.at[pl.ds(wid * _IDX_W, _IDX_W)], idx_v)
        copies = []
        for j in range(_NCHUNK):
            copies.append(
                pltpu.async_copy(
                    table_hbm.at[idx_v.at[pl.ds(j * _CHUNK, _CHUNK)]],
                    rows_v.at[pl.ds(j * _CHUNK, _CHUNK)],
                    sem,
                )
            )
        for c in copies:
            c.wait()
        inv = jnp.float32(1.0 / _CTX)
        for r in range(_ROWS_W):
            acc = rows_v[r * _CTX]
            for t in range(1, _CTX):
                acc = acc + rows_v[r * _CTX + t]
            out_v[r] = acc * inv
        pltpu.sync_copy(out_v, out_hbm.at[pl.ds(wid * _ROWS_W, _ROWS_W)])

    return pool_sc


_PCOL = 8192
_PGRID = (_VOCAB + _PCOL - 1) // _PCOL


def _pack_body(wt_ref, out_ref):
    y = wt_ref[...].T.reshape(_PCOL // 8, 8, _EMBED)
    for s in range(8):
        out_ref[:, s * _EMBED:(s + 1) * _EMBED] = y[:, s, :]


def _pack_table(Wt_view):
    return pl.pallas_call(
        _pack_body,
        grid=(_PGRID,),
        in_specs=[pl.BlockSpec((_EMBED, _PCOL), lambda i: (0, i))],
        out_specs=pl.BlockSpec((_PCOL * _EMBED // 128, 128), lambda i: (i, 0)),
        out_shape=jax.ShapeDtypeStruct((_VOCAB * _EMBED // 128, 128),
                                       jnp.float32),
    )(Wt_view)


_VB = 4096
_GRID = (_VOCAB + _VB - 1) // _VB


def _proj_body(wt_ref, emb_ref, b_ref, out_ref):
    out_ref[...] = (
        lax.dot_general(
            wt_ref[...],
            emb_ref[...],
            dimension_numbers=(((0,), (1,)), ((), ())),
            preferred_element_type=jnp.float32,
        )
        + b_ref[...].T
    )


def _project_t(pooled, Wt, b2):
    return pl.pallas_call(
        _proj_body,
        grid=(_GRID,),
        in_specs=[
            pl.BlockSpec((_EMBED, _VB), lambda i: (0, i)),
            pl.BlockSpec((_BATCH, _EMBED), lambda i: (0, 0)),
            pl.BlockSpec((1, _VB), lambda i: (0, i)),
        ],
        out_specs=pl.BlockSpec((_VB, _BATCH), lambda i: (i, 0)),
        out_shape=jax.ShapeDtypeStruct((_VOCAB, _BATCH), jnp.float32),
    )(Wt, pooled, b2)


def kernel(context_words, target_word, emb_table, W, b):
    del target_word
    idx = context_words.astype(jnp.int32).reshape(_BATCH * _CTX)
    packed = _pack_table(emb_table.T)
    pooled = _build_pool_sc()(idx, packed.reshape(_VOCAB, _EMBED))
    logits_t = _project_t(pooled, W.T, b.reshape(1, _VOCAB))
    return logits_t.T
